# baseline (speedup 1.0000x reference)
import jax
import jax.numpy as jnp
from jax.experimental import pallas as pl
from jax.experimental.pallas import tpu as pltpu

DIM_CODES, BOOK_SIZE, EMB = 128, 1024, 64
CW_DIM = DIM_CODES * EMB
Z_DIM = 256
H_DIM = 512
N_PSEUDO = 400
BATCH = 64

K_EMB = 64
N_BLK = 1024
DC_BLK = 8


def _enc_kernel(x3_ref, pp_ref, We1_ref, be1_ref, We2_ref, be2_ref, eps_ref,
                Wd1_ref, bd1_ref,
                mu_ref, lv_ref, pmu_ref, plv_ref, z_ref, hd_ref,
                acc_ref, xbt_ref):
    k = pl.program_id(0)
    nk = pl.num_programs(0)

    @pl.when(k == 0)
    def _():
        xfull = x3_ref[...].reshape(BATCH, DIM_CODES, EMB)
        for kb in range(EMB // K_EMB):
            xbt_ref[kb] = jnp.swapaxes(
                xfull[:, :, kb * K_EMB:(kb + 1) * K_EMB], 1, 2
            ).reshape(BATCH, K_EMB * DIM_CODES)

    xb = xbt_ref[k]
    ppf = pp_ref[...].reshape(N_PSEUDO, K_EMB * DIM_CODES)
    xa = jnp.concatenate([xb, ppf], axis=0)
    part = jnp.dot(xa, We1_ref[...], preferred_element_type=jnp.float32)

    @pl.when(k == 0)
    def _():
        acc_ref[...] = part

    @pl.when(k > 0)
    def _():
        acc_ref[...] += part

    @pl.when(k == nk - 1)
    def _():
        h = jnp.maximum(acc_ref[...] + be1_ref[...], 0.0)
        enc = (jnp.dot(h, We2_ref[...], preferred_element_type=jnp.float32)
               + be2_ref[...])
        mu = enc[:BATCH, :Z_DIM]
        lv = enc[:BATCH, Z_DIM:]
        mu_ref[...] = mu
        lv_ref[...] = lv
        pmu_ref[...] = enc[BATCH:, :Z_DIM]
        plv_ref[...] = enc[BATCH:, Z_DIM:]
        z = eps_ref[...] * jnp.exp(0.5 * lv) + mu
        z_ref[...] = z
        hd_ref[...] = jnp.maximum(
            jnp.dot(z, Wd1_ref[...], preferred_element_type=jnp.float32)
            + bd1_ref[...], 0.0)


def _dist_kernel(hd_ref, Wd2_ref, bd2_ref, book_ref,
                 recon_ref, d_ref, idx_ref, scr_ref):
    i = pl.program_id(0)
    ni = pl.num_programs(0)
    xr = (jnp.dot(hd_ref[...], Wd2_ref[...],
                  preferred_element_type=jnp.float32)
          + bd2_ref[...])
    recon_ref[...] = xr
    iota = jax.lax.broadcasted_iota(jnp.int32, (BATCH, BOOK_SIZE), 1)
    djs = []
    for j in range(DC_BLK):
        xj = xr[:, j * EMB:(j + 1) * EMB]
        bj = book_ref[j]
        b2 = jnp.sum(bj * bj, axis=1)[None, :]
        x2 = jnp.sum(xj * xj, axis=1, keepdims=True)
        cross = jax.lax.dot_general(
            xj, bj, (((1,), (1,)), ((), ())),
            preferred_element_type=jnp.float32)
        dj = x2 + b2 - 2.0 * cross
        djs.append(dj)
        m = jnp.min(dj, axis=1, keepdims=True)
        idxj = jnp.min(jnp.where(dj <= m, iota, BOOK_SIZE),
                       axis=1, keepdims=True).astype(jnp.int32)
        scr_ref[i, :, j:j + 1] = idxj
    d_ref[...] = jnp.stack(djs, axis=1)

    @pl.when(i == ni - 1)
    def _():
        v = scr_ref[...]
        idx_ref[...] = jnp.transpose(v, (1, 0, 2)).reshape(BATCH, DIM_CODES)


def kernel(x, pseudo_inputs, codebook, We1, be1, We2, be2, Wd1, bd1, Wd2, bd2):
    B = x.shape[0]
    R = B + N_PSEUDO

    pp = pseudo_inputs
    be1r = be1.reshape(1, H_DIM)
    be2r = be2.reshape(1, 2 * Z_DIM)
    bd1r = bd1.reshape(1, H_DIM)
    bd2r = bd2.reshape(1, CW_DIM)

    eps = jax.random.normal(jax.random.key(42), (B, Z_DIM), dtype=jnp.float32)

    nk = EMB // K_EMB
    outs = pl.pallas_call(
        _enc_kernel,
        grid=(nk,),
        in_specs=[
            pl.BlockSpec((B, CW_DIM), lambda k: (0, 0)),
            pl.BlockSpec((N_PSEUDO, K_EMB, DIM_CODES), lambda k: (0, k, 0)),
            pl.BlockSpec((K_EMB * DIM_CODES, H_DIM), lambda k: (k, 0)),
            pl.BlockSpec((1, H_DIM), lambda k: (0, 0)),
            pl.BlockSpec((H_DIM, 2 * Z_DIM), lambda k: (0, 0)),
            pl.BlockSpec((1, 2 * Z_DIM), lambda k: (0, 0)),
            pl.BlockSpec((B, Z_DIM), lambda k: (0, 0)),
            pl.BlockSpec((Z_DIM, H_DIM), lambda k: (0, 0)),
            pl.BlockSpec((1, H_DIM), lambda k: (0, 0)),
        ],
        out_specs=[
            pl.BlockSpec((B, Z_DIM), lambda k: (0, 0)),
            pl.BlockSpec((B, Z_DIM), lambda k: (0, 0)),
            pl.BlockSpec((N_PSEUDO, Z_DIM), lambda k: (0, 0)),
            pl.BlockSpec((N_PSEUDO, Z_DIM), lambda k: (0, 0)),
            pl.BlockSpec((B, Z_DIM), lambda k: (0, 0)),
            pl.BlockSpec((B, H_DIM), lambda k: (0, 0)),
        ],
        out_shape=[
            jax.ShapeDtypeStruct((B, Z_DIM), jnp.float32),
            jax.ShapeDtypeStruct((B, Z_DIM), jnp.float32),
            jax.ShapeDtypeStruct((N_PSEUDO, Z_DIM), jnp.float32),
            jax.ShapeDtypeStruct((N_PSEUDO, Z_DIM), jnp.float32),
            jax.ShapeDtypeStruct((B, Z_DIM), jnp.float32),
            jax.ShapeDtypeStruct((B, H_DIM), jnp.float32),
        ],
        scratch_shapes=[pltpu.VMEM((R, H_DIM), jnp.float32),
                        pltpu.VMEM((EMB // K_EMB, B, K_EMB * DIM_CODES),
                                   jnp.float32)],
    )(x, pp, We1, be1r, We2, be2r, eps, Wd1, bd1r)
    mu, log_var, pseudo_mu, pseudo_log_var, z, hd = outs

    ni = DIM_CODES // DC_BLK
    cw_recon, d2, idx2 = pl.pallas_call(
        _dist_kernel,
        grid=(ni,),
        in_specs=[
            pl.BlockSpec((B, H_DIM), lambda i: (0, 0)),
            pl.BlockSpec((H_DIM, DC_BLK * EMB), lambda i: (0, i)),
            pl.BlockSpec((1, DC_BLK * EMB), lambda i: (0, i)),
            pl.BlockSpec((DC_BLK, BOOK_SIZE, EMB), lambda i: (i, 0, 0)),
        ],
        out_specs=[
            pl.BlockSpec((B, DC_BLK * EMB), lambda i: (0, i)),
            pl.BlockSpec((B, DC_BLK, BOOK_SIZE), lambda i: (0, i, 0)),
            pl.BlockSpec((B, DIM_CODES), lambda i: (0, 0)),
        ],
        out_shape=[
            jax.ShapeDtypeStruct((B, CW_DIM), jnp.float32),
            jax.ShapeDtypeStruct((B, DIM_CODES, BOOK_SIZE), jnp.float32),
            jax.ShapeDtypeStruct((B, DIM_CODES), jnp.int32),
        ],
        scratch_shapes=[pltpu.VMEM((ni, B, DC_BLK), jnp.int32)],
    )(hd, Wd2, bd2r, codebook)

    cw_dist = d2
    idx = idx2.reshape(B * DIM_CODES, 1)

    return (cw_recon, cw_dist, idx, mu, log_var,
            pseudo_mu, pseudo_log_var, z)


# speedup vs baseline: 1.1131x; 1.1131x over previous
import jax
import jax.numpy as jnp
from jax.experimental import pallas as pl
from jax.experimental.pallas import tpu as pltpu

DIM_CODES, BOOK_SIZE, EMB = 128, 1024, 64
CW_DIM = DIM_CODES * EMB
Z_DIM = 256
H_DIM = 512
N_PSEUDO = 400
BATCH = 64

K_EMB = 64
N_BLK = 1024
DC_BLK = 8


def _enc_kernel(x3_ref, pp_ref, We1_ref, be1_ref, We2_ref, be2_ref, eps_ref,
                Wd1_ref, bd1_ref,
                mu_ref, lv_ref, pmu_ref, plv_ref, z_ref, hd_ref,
                acc_ref, xbt_ref):
    k = pl.program_id(0)
    nk = pl.num_programs(0)

    @pl.when(k == 0)
    def _():
        xfull = x3_ref[...].reshape(BATCH, DIM_CODES, EMB)
        for kb in range(EMB // K_EMB):
            xbt_ref[kb] = jnp.swapaxes(
                xfull[:, :, kb * K_EMB:(kb + 1) * K_EMB], 1, 2
            ).reshape(BATCH, K_EMB * DIM_CODES)

    xb = xbt_ref[k]
    ppf = pp_ref[...].reshape(N_PSEUDO, K_EMB * DIM_CODES)
    xa = jnp.concatenate([xb, ppf], axis=0)
    part = jnp.dot(xa, We1_ref[...], preferred_element_type=jnp.float32)

    @pl.when(k == 0)
    def _():
        acc_ref[...] = part

    @pl.when(k > 0)
    def _():
        acc_ref[...] += part

    @pl.when(k == nk - 1)
    def _():
        h = jnp.maximum(acc_ref[...] + be1_ref[...], 0.0)
        enc = (jnp.dot(h, We2_ref[...], preferred_element_type=jnp.float32)
               + be2_ref[...])
        mu = enc[:BATCH, :Z_DIM]
        lv = enc[:BATCH, Z_DIM:]
        mu_ref[...] = mu
        lv_ref[...] = lv
        pmu_ref[...] = enc[BATCH:, :Z_DIM]
        plv_ref[...] = enc[BATCH:, Z_DIM:]
        z = eps_ref[...] * jnp.exp(0.5 * lv) + mu
        z_ref[...] = z
        hd_ref[...] = jnp.maximum(
            jnp.dot(z, Wd1_ref[...], preferred_element_type=jnp.float32)
            + bd1_ref[...], 0.0)


def _dist_kernel(hd_ref, Wd2_ref, bd2_ref, book_ref,
                 recon_ref, d_ref, idx_ref, scr_ref):
    i = pl.program_id(0)
    ni = pl.num_programs(0)
    xr = (jnp.dot(hd_ref[...], Wd2_ref[...],
                  preferred_element_type=jnp.float32)
          + bd2_ref[...])
    recon_ref[...] = xr
    iota = jax.lax.broadcasted_iota(jnp.int32, (BATCH, BOOK_SIZE), 1)
    djs = []
    for j in range(DC_BLK):
        xj = xr[:, j * EMB:(j + 1) * EMB]
        bj = book_ref[j * BOOK_SIZE:(j + 1) * BOOK_SIZE, :]
        b2 = jnp.sum(bj * bj, axis=1)[None, :]
        x2 = jnp.sum(xj * xj, axis=1, keepdims=True)
        cross = jax.lax.dot_general(
            xj, bj, (((1,), (1,)), ((), ())),
            preferred_element_type=jnp.float32)
        dj = x2 + b2 - 2.0 * cross
        djs.append(dj)
        m = jnp.min(dj, axis=1, keepdims=True)
        idxj = jnp.min(jnp.where(dj <= m, iota, BOOK_SIZE),
                       axis=1, keepdims=True).astype(jnp.int32)
        scr_ref[i, :, j:j + 1] = idxj
    d_ref[...] = jnp.stack(djs, axis=1)

    @pl.when(i == ni - 1)
    def _():
        v = scr_ref[...]
        idx_ref[...] = jnp.transpose(v, (1, 0, 2)).reshape(BATCH, DIM_CODES)


def kernel(x, pseudo_inputs, codebook, We1, be1, We2, be2, Wd1, bd1, Wd2, bd2):
    B = x.shape[0]
    R = B + N_PSEUDO

    pp = pseudo_inputs
    be1r = be1.reshape(1, H_DIM)
    be2r = be2.reshape(1, 2 * Z_DIM)
    bd1r = bd1.reshape(1, H_DIM)
    bd2r = bd2.reshape(1, CW_DIM)

    eps = jax.random.normal(jax.random.key(42), (B, Z_DIM), dtype=jnp.float32)

    nk = EMB // K_EMB
    outs = pl.pallas_call(
        _enc_kernel,
        grid=(nk,),
        in_specs=[
            pl.BlockSpec((B, CW_DIM), lambda k: (0, 0)),
            pl.BlockSpec((N_PSEUDO, K_EMB, DIM_CODES), lambda k: (0, k, 0)),
            pl.BlockSpec((K_EMB * DIM_CODES, H_DIM), lambda k: (k, 0)),
            pl.BlockSpec((1, H_DIM), lambda k: (0, 0)),
            pl.BlockSpec((H_DIM, 2 * Z_DIM), lambda k: (0, 0)),
            pl.BlockSpec((1, 2 * Z_DIM), lambda k: (0, 0)),
            pl.BlockSpec((B, Z_DIM), lambda k: (0, 0)),
            pl.BlockSpec((Z_DIM, H_DIM), lambda k: (0, 0)),
            pl.BlockSpec((1, H_DIM), lambda k: (0, 0)),
        ],
        out_specs=[
            pl.BlockSpec((B, Z_DIM), lambda k: (0, 0)),
            pl.BlockSpec((B, Z_DIM), lambda k: (0, 0)),
            pl.BlockSpec((N_PSEUDO, Z_DIM), lambda k: (0, 0)),
            pl.BlockSpec((N_PSEUDO, Z_DIM), lambda k: (0, 0)),
            pl.BlockSpec((B, Z_DIM), lambda k: (0, 0)),
            pl.BlockSpec((B, H_DIM), lambda k: (0, 0)),
        ],
        out_shape=[
            jax.ShapeDtypeStruct((B, Z_DIM), jnp.float32),
            jax.ShapeDtypeStruct((B, Z_DIM), jnp.float32),
            jax.ShapeDtypeStruct((N_PSEUDO, Z_DIM), jnp.float32),
            jax.ShapeDtypeStruct((N_PSEUDO, Z_DIM), jnp.float32),
            jax.ShapeDtypeStruct((B, Z_DIM), jnp.float32),
            jax.ShapeDtypeStruct((B, H_DIM), jnp.float32),
        ],
        scratch_shapes=[pltpu.VMEM((R, H_DIM), jnp.float32),
                        pltpu.VMEM((EMB // K_EMB, B, K_EMB * DIM_CODES),
                                   jnp.float32)],
    )(x, pp, We1, be1r, We2, be2r, eps, Wd1, bd1r)
    mu, log_var, pseudo_mu, pseudo_log_var, z, hd = outs

    book2d = codebook.reshape(DIM_CODES * BOOK_SIZE, EMB)
    ni = DIM_CODES // DC_BLK
    cw_recon, d2, idx2 = pl.pallas_call(
        _dist_kernel,
        grid=(ni,),
        in_specs=[
            pl.BlockSpec((B, H_DIM), lambda i: (0, 0)),
            pl.BlockSpec((H_DIM, DC_BLK * EMB), lambda i: (0, i)),
            pl.BlockSpec((1, DC_BLK * EMB), lambda i: (0, i)),
            pl.BlockSpec((DC_BLK * BOOK_SIZE, EMB), lambda i: (i, 0)),
        ],
        out_specs=[
            pl.BlockSpec((B, DC_BLK * EMB), lambda i: (0, i)),
            pl.BlockSpec((B, DC_BLK, BOOK_SIZE), lambda i: (0, i, 0)),
            pl.BlockSpec((B, DIM_CODES), lambda i: (0, 0)),
        ],
        out_shape=[
            jax.ShapeDtypeStruct((B, CW_DIM), jnp.float32),
            jax.ShapeDtypeStruct((B, DIM_CODES, BOOK_SIZE), jnp.float32),
            jax.ShapeDtypeStruct((B, DIM_CODES), jnp.int32),
        ],
        scratch_shapes=[pltpu.VMEM((ni, B, DC_BLK), jnp.int32)],
    )(hd, Wd2, bd2r, book2d)

    cw_dist = d2
    idx = idx2.reshape(B * DIM_CODES, 1)

    return (cw_recon, cw_dist, idx, mu, log_var,
            pseudo_mu, pseudo_log_var, z)
